# baseline (speedup 1.0000x reference)
import functools

import jax
import jax.numpy as jnp
from jax.experimental import pallas as pl
from jax.experimental.pallas import tpu as pltpu


def _matmul_cast_kernel(x_ref, w_ref, out_ref):
    xb = x_ref[...].astype(jnp.bfloat16)
    out_ref[...] = jnp.dot(
        xb, w_ref[...], preferred_element_type=jnp.float32
    ).astype(jnp.bfloat16)


def _gcn_layer_kernel(adj_ref, sup_ref, b_ref, w_ref, bout_ref, out_ref,
                      *, out_dtype, add_bout):
    adj_b = adj_ref[...].astype(jnp.bfloat16)
    acc = jnp.dot(adj_b, sup_ref[...], preferred_element_type=jnp.float32)
    h = jnp.maximum(acc + b_ref[...], 0.0).astype(jnp.bfloat16)
    res = jnp.dot(h, w_ref[...], preferred_element_type=jnp.float32)
    if add_bout:
        res = res + bout_ref[...]
    out_ref[...] = res.astype(out_dtype)


def _pad2(a, rows, cols):
    r, c = a.shape
    if r == rows and c == cols:
        return a
    return jnp.pad(a, ((0, rows - r), (0, cols - c)))


def _gcn_layer(adj, sup, b, w, bout, out_dtype, add_bout, bm):
    n = adj.shape[0]
    k = adj.shape[1]
    hin = sup.shape[1]
    hout = w.shape[1]
    fn = functools.partial(_gcn_layer_kernel, out_dtype=out_dtype,
                           add_bout=add_bout)
    return pl.pallas_call(
        fn,
        grid=(pl.cdiv(n, bm),),
        in_specs=[
            pl.BlockSpec((bm, k), lambda i: (i, 0)),
            pl.BlockSpec((k, hin), lambda i: (0, 0)),
            pl.BlockSpec((1, hin), lambda i: (0, 0)),
            pl.BlockSpec((hin, hout), lambda i: (0, 0)),
            pl.BlockSpec((1, hout), lambda i: (0, 0)),
        ],
        out_specs=pl.BlockSpec((bm, hout), lambda i: (i, 0)),
        out_shape=jax.ShapeDtypeStruct((n, hout), out_dtype),
        compiler_params=pltpu.CompilerParams(
            dimension_semantics=("parallel",)),
    )(adj, sup, b, w, bout)


def kernel(x, adj, W1, b1, W2, b2, Wfc, bfc):
    N, F = x.shape
    H1 = W1.shape[1]
    H2 = W2.shape[1]
    C = Wfc.shape[1]
    H1p = ((H1 + 127) // 128) * 128
    H2p = ((H2 + 127) // 128) * 128
    Cp = ((C + 127) // 128) * 128

    BM1 = 1000 if N % 1000 == 0 else min(N, 1024)
    W1p = _pad2(W1, F, H1p).astype(jnp.bfloat16)
    support1 = pl.pallas_call(
        _matmul_cast_kernel,
        grid=(pl.cdiv(N, BM1),),
        in_specs=[
            pl.BlockSpec((BM1, F), lambda i: (i, 0)),
            pl.BlockSpec((F, H1p), lambda i: (0, 0)),
        ],
        out_specs=pl.BlockSpec((BM1, H1p), lambda i: (i, 0)),
        out_shape=jax.ShapeDtypeStruct((N, H1p), jnp.bfloat16),
        compiler_params=pltpu.CompilerParams(
            dimension_semantics=("parallel",)),
    )(x, W1p)

    BM = 400 if N % 400 == 0 else min(N, 512)
    b1p = _pad2(b1[None, :], 1, H1p)
    W2p = _pad2(W2, H1p, H2p).astype(jnp.bfloat16)
    zero_bias = jnp.zeros((1, H2p), jnp.float32)
    support2 = _gcn_layer(adj, support1, b1p, W2p, zero_bias,
                          jnp.bfloat16, False, BM)

    b2p = _pad2(b2[None, :], 1, H2p)
    Wfcp = _pad2(Wfc, H2p, Cp).astype(jnp.bfloat16)
    bfcp = _pad2(bfc[None, :], 1, Cp)
    logits_p = _gcn_layer(adj, support2, b2p, Wfcp, bfcp,
                          jnp.float32, True, BM)

    return logits_p[:, :C]
